# Initial kernel scaffold; baseline (speedup 1.0000x reference)
import jax
import jax.numpy as jnp
from jax.experimental import pallas as pl


def kernel(x):
    raise NotImplementedError("write your pallas kernel here")


# speedup vs baseline: 3.4496x; 3.4496x over previous
import jax
import jax.numpy as jnp
from jax import lax
from jax.experimental import pallas as pl
from jax.experimental.pallas import tpu as pltpu
from jax.experimental.pallas import tpu_sc as plsc

_B = 4096
_F = 50 * 128
_NC, _NS = 2, 16
_NW = _NC * _NS
_RPW = _B // _NW
_CH = 8
_NCHUNK = _RPW // _CH


def _sc_roll_body(x_ref, o_ref, buf):
    wid = lax.axis_index("s") * _NC + lax.axis_index("c")
    base = wid * _RPW

    def chunk(c, carry):
        s = (base + c * _CH) * _F
        is_wrap = base + c * _CH + _CH >= _B

        @pl.when(jnp.logical_not(is_wrap))
        def _():
            pltpu.sync_copy(x_ref.at[pl.ds(s + _F, _CH * _F)], buf)
            pltpu.sync_copy(buf, o_ref.at[pl.ds(s, _CH * _F)])

        @pl.when(is_wrap)
        def _():
            pltpu.sync_copy(x_ref.at[pl.ds(s + _F, (_CH - 1) * _F)],
                            buf.at[pl.ds(0, (_CH - 1) * _F)])
            pltpu.sync_copy(x_ref.at[pl.ds(0, _F)],
                            buf.at[pl.ds((_CH - 1) * _F, _F)])
            pltpu.sync_copy(buf, o_ref.at[pl.ds(s, _CH * _F)])

        return carry

    lax.fori_loop(0, _NCHUNK, chunk, 0)


def kernel(x):
    x1 = x.reshape(_B * _F)
    out = pl.kernel(
        _sc_roll_body,
        out_type=jax.ShapeDtypeStruct((_B * _F,), jnp.float32),
        mesh=plsc.VectorSubcoreMesh(core_axis_name="c", subcore_axis_name="s"),
        scratch_types=[pltpu.VMEM((_CH * _F,), jnp.float32)],
    )(x1)
    return out.reshape(x.shape)
